# baseline (speedup 1.0000x reference)
import functools

import jax
import jax.numpy as jnp
from jax import lax
from jax.experimental import pallas as pl
from jax.experimental.pallas import tpu as pltpu
from jax.experimental.pallas import tpu_sc as plsc

MAX_LENGTH = 40
EMBED_DIM = 64
BATCH = 16384
ROWS = BATCH * MAX_LENGTH
NUM_WORKERS = 32
ROWS_PER_WORKER = ROWS // NUM_WORKERS
CHUNK = 640
GATHERS_PER_CHUNK = CHUNK // 128
CHUNKS_PER_WORKER = ROWS_PER_WORKER // CHUNK
BATCHES_PER_CHUNK = CHUNK // MAX_LENGTH

_mesh = plsc.VectorSubcoreMesh(core_axis_name="c", subcore_axis_name="s")


@functools.partial(
    pl.kernel,
    mesh=_mesh,
    compiler_params=pltpu.CompilerParams(use_tc_tiling_on_sc=False),
    out_type=jax.ShapeDtypeStruct((ROWS, EMBED_DIM), jnp.float32),
    scratch_types=[
        pltpu.VMEM((ROWS_PER_WORKER // 128, 128), jnp.int32),
        pltpu.VMEM((CHUNK, EMBED_DIM), jnp.float32),
        pltpu.VMEM((MAX_LENGTH, EMBED_DIM), jnp.float32),
        pltpu.SemaphoreType.DMA,
    ],
)
def _embed(x_hbm, tok_hbm, pos_hbm, out_hbm, idx_v, rows_v, pos_v, sem):
    wid = lax.axis_index("s") * 2 + lax.axis_index("c")
    base = wid * ROWS_PER_WORKER
    pltpu.sync_copy(pos_hbm, pos_v)
    pltpu.sync_copy(x_hbm.at[pl.ds(wid * (ROWS_PER_WORKER // 128),
                                   ROWS_PER_WORKER // 128)], idx_v)

    def chunk_body(ci, carry):
        r0 = base + ci * CHUNK
        copies = [
            pltpu.async_copy(
                tok_hbm.at[idx_v.at[ci * GATHERS_PER_CHUNK + j]],
                rows_v.at[pl.ds(j * 128, 128)],
                sem,
            )
            for j in range(GATHERS_PER_CHUNK)
        ]
        for c in copies:
            c.wait()

        def l_body(l, carry2):
            pvs = [pos_v[l, pl.ds(k * 16, 16)] for k in range(4)]

            def b_body(b, carry3):
                r = b * MAX_LENGTH + l
                for k in range(4):
                    plsc.addupdate(rows_v.at[r, pl.ds(k * 16, 16)], pvs[k])
                return carry3

            return lax.fori_loop(0, BATCHES_PER_CHUNK, b_body, carry2)

        lax.fori_loop(0, MAX_LENGTH, l_body, 0)
        pltpu.sync_copy(rows_v, out_hbm.at[pl.ds(r0, CHUNK)])
        return carry

    lax.fori_loop(0, CHUNKS_PER_WORKER, chunk_body, 0)


def kernel(x, token_table, position_table):
    x_flat = x.reshape(ROWS // 128, 128)
    out = _embed(x_flat, token_table, position_table)
    return out.reshape(BATCH, MAX_LENGTH, EMBED_DIM)
